# Initial kernel scaffold; baseline (speedup 1.0000x reference)
import jax
import jax.numpy as jnp
from jax.experimental import pallas as pl


def kernel(x, edge_index, W1, b1, W2, b2):
    raise NotImplementedError("write your pallas kernel here")


# speedup vs baseline: 8.0352x; 8.0352x over previous
import functools

import jax
import jax.numpy as jnp
from jax import lax
from jax.experimental import pallas as pl
from jax.experimental.pallas import tpu as pltpu
from jax.experimental.pallas import tpu_sc as plsc

N = 10000
E = 320000
D = 128

NC = 2
NS = 16
NW = NC * NS
K = 128
C = 80
EP = NW * C * K
EPT = C * K
NP = 10240
RPT = NP // NS
CW = 16

_mesh = plsc.VectorSubcoreMesh(core_axis_name="c", subcore_axis_name="s")


@functools.partial(
    pl.kernel,
    out_type=jax.ShapeDtypeStruct((NC, NP, D), jnp.float32),
    mesh=_mesh,
    scratch_types=[
        pltpu.VMEM((C, K), jnp.int32),
        pltpu.VMEM((K, D), jnp.float32),
        pltpu.VMEM_SHARED((NP, D), jnp.float32),
    ],
)
def _sc_count(dst_hbm, ones_hbm, zeros_hbm, out_hbm, idx_d, ones_v, acc_sh):
    cid = lax.axis_index("c")
    sid = lax.axis_index("s")
    wid = sid * NC + cid
    pltpu.sync_copy(dst_hbm.at[wid], idx_d)
    pltpu.sync_copy(ones_hbm, ones_v)
    pltpu.sync_copy(zeros_hbm, acc_sh.at[pl.ds(sid * RPT, RPT)])
    plsc.subcore_barrier()

    def body(j, carry):
        pltpu.sync_copy(ones_v, acc_sh.at[idx_d.at[j]], add=True)
        return carry

    lax.fori_loop(0, C, body, 0)
    plsc.subcore_barrier()
    pltpu.sync_copy(acc_sh.at[pl.ds(sid * RPT, RPT)],
                    out_hbm.at[cid, pl.ds(sid * RPT, RPT)])


@functools.partial(
    pl.kernel,
    out_type=jax.ShapeDtypeStruct((NC, NP, D), jnp.float32),
    mesh=_mesh,
    scratch_types=[
        pltpu.VMEM((C, K), jnp.int32),
        pltpu.VMEM((C, K), jnp.int32),
        pltpu.VMEM((K, D), jnp.float32),
        pltpu.VMEM_SHARED((NP, D), jnp.float32),
    ],
)
def _sc_agg(hs_hbm, src_hbm, dst_hbm, zeros_hbm, out_hbm,
            idx_s, idx_d, rows, acc_sh):
    cid = lax.axis_index("c")
    sid = lax.axis_index("s")
    wid = sid * NC + cid
    pltpu.sync_copy(src_hbm.at[wid], idx_s)
    pltpu.sync_copy(dst_hbm.at[wid], idx_d)
    pltpu.sync_copy(zeros_hbm, acc_sh.at[pl.ds(sid * RPT, RPT)])
    plsc.subcore_barrier()

    def body(j, carry):
        pltpu.sync_copy(hs_hbm.at[idx_s.at[j]], rows)
        pltpu.sync_copy(rows, acc_sh.at[idx_d.at[j]], add=True)
        return carry

    lax.fori_loop(0, C, body, 0)
    plsc.subcore_barrier()
    pltpu.sync_copy(acc_sh.at[pl.ds(sid * RPT, RPT)],
                    out_hbm.at[cid, pl.ds(sid * RPT, RPT)])


_R = 1000


def _t1_body(x_ref, w_ref, cnt_ref, hs_ref, c_ref):
    cnt = cnt_ref[0, :, 0] + cnt_ref[1, :, 0]
    c = lax.rsqrt(cnt + 1.0)
    cb = jnp.broadcast_to(c[:, None], (_R, D))
    h = jnp.dot(x_ref[...], w_ref[...], preferred_element_type=jnp.float32)
    hs_ref[...] = cb * h
    c_ref[...] = cb


def _t2_body(acc_ref, hs_ref, c_ref, b_ref, w_ref, out_ref):
    agg = acc_ref[0] + acc_ref[1] + hs_ref[...]
    t = jnp.maximum(c_ref[...] * agg + b_ref[...], 0.0)
    out_ref[...] = c_ref[...] * jnp.dot(
        t, w_ref[...], preferred_element_type=jnp.float32)


def _t3_body(acc_ref, hs_ref, c_ref, b_ref, out_ref):
    agg = acc_ref[0] + acc_ref[1] + hs_ref[...]
    out_ref[...] = jnp.maximum(c_ref[...] * agg + b_ref[...], 0.0)


_row_spec = pl.BlockSpec((_R, D), lambda i: (i, 0))
_acc_spec = pl.BlockSpec((NC, _R, D), lambda i: (0, i, 0))
_cnt_spec = pl.BlockSpec((NC, _R, D), lambda i: (0, i, 0))
_w_spec = pl.BlockSpec((D, D), lambda i: (0, 0))
_b_spec = pl.BlockSpec((1, D), lambda i: (0, 0))

_t1 = pl.pallas_call(
    _t1_body,
    grid=(N // _R,),
    in_specs=[_row_spec, _w_spec, _cnt_spec],
    out_specs=[_row_spec, _row_spec],
    out_shape=[jax.ShapeDtypeStruct((N, D), jnp.float32),
               jax.ShapeDtypeStruct((N, D), jnp.float32)],
)

_t2 = pl.pallas_call(
    _t2_body,
    grid=(N // _R,),
    in_specs=[_acc_spec, _row_spec, _row_spec, _b_spec, _w_spec],
    out_specs=_row_spec,
    out_shape=jax.ShapeDtypeStruct((N, D), jnp.float32),
)

_t3 = pl.pallas_call(
    _t3_body,
    grid=(N // _R,),
    in_specs=[_acc_spec, _row_spec, _row_spec, _b_spec],
    out_specs=_row_spec,
    out_shape=jax.ShapeDtypeStruct((N, D), jnp.float32),
)


def kernel(x, edge_index, W1, b1, W2, b2):
    pad = EP - E
    src = jnp.concatenate(
        [edge_index[0], jnp.zeros((pad,), jnp.int32)]).reshape(NW, C, K)
    dst = jnp.concatenate(
        [edge_index[1], jnp.full((pad,), N, jnp.int32)]).reshape(NW, C, K)
    zeros_d = jnp.zeros((RPT, D), jnp.float32)
    ones_d = jnp.ones((K, D), jnp.float32)

    cnt = _sc_count(dst, ones_d, zeros_d)
    hs1, cbc = _t1(x, W1, cnt)
    acc1 = _sc_agg(hs1, src, dst, zeros_d)
    hs2 = _t2(acc1, hs1, cbc, b1.reshape(1, D), W2)
    acc2 = _sc_agg(hs2, src, dst, zeros_d)
    return _t3(acc2, hs2, cbc, b2.reshape(1, D))
